# Initial kernel scaffold; baseline (speedup 1.0000x reference)
import jax
import jax.numpy as jnp
from jax.experimental import pallas as pl


def kernel(x, edge_index, batch, batch_size, W1, b1, g1, be1, W2, b2, g2, be2, W3, b3, g3, be3, Wl1, bl1, g4, be4, Wl2, bl2):
    raise NotImplementedError("write your pallas kernel here")


# speedup vs baseline: 6.9599x; 6.9599x over previous
import functools

import jax
import jax.numpy as jnp
from jax import lax
from jax.experimental import pallas as pl
from jax.experimental.pallas import tpu as pltpu
from jax.experimental.pallas import tpu_sc as plsc

N = 10000
E = 320000
NPAD = 10112
RPT = NPAD // 16
NW = 32
CHUNK = 128
NCHUNK = 80
EPW = NCHUNK * CHUNK
EPAD = NW * EPW
D = 128
EPS = 1e-5

_MESH = dict(core_axis_name="c", subcore_axis_name="s")


@functools.lru_cache(maxsize=None)
def _make_agg():
    @functools.partial(
        pl.kernel,
        mesh=plsc.VectorSubcoreMesh(**_MESH),
        out_type=jax.ShapeDtypeStruct((2, NPAD, D), jnp.float32),
        scratch_types=[
            pltpu.VMEM((CHUNK,), jnp.int32),
            pltpu.VMEM((CHUNK,), jnp.int32),
            pltpu.VMEM((CHUNK, D), jnp.float32),
            pltpu.VMEM_SHARED((NPAD, D), jnp.float32),
            pltpu.SemaphoreType.DMA,
        ],
    )
    def agg(table, src1d, dst1d, zeros, out, src_v, dst_v, rows_v, acc, sem):
        c = lax.axis_index("c")
        s = lax.axis_index("s")
        wid = c * 16 + s
        pltpu.sync_copy(zeros.at[pl.ds(s * RPT, RPT)],
                        acc.at[pl.ds(s * RPT, RPT)])
        plsc.subcore_barrier()

        def body(j, _):
            base = (wid * NCHUNK + j) * CHUNK
            pltpu.sync_copy(src1d.at[pl.ds(base, CHUNK)], src_v)
            pltpu.sync_copy(dst1d.at[pl.ds(base, CHUNK)], dst_v)
            pltpu.async_copy(table.at[src_v], rows_v, sem).wait()
            pltpu.sync_copy(rows_v, acc.at[dst_v], add=True)
            return ()

        lax.fori_loop(0, NCHUNK, body, ())
        plsc.subcore_barrier()
        pltpu.sync_copy(acc.at[pl.ds(s * RPT, RPT)],
                        out.at[c, pl.ds(s * RPT, RPT)])

    return agg


@functools.lru_cache(maxsize=None)
def _make_deg():
    @functools.partial(
        pl.kernel,
        mesh=plsc.VectorSubcoreMesh(**_MESH),
        compiler_params=pltpu.CompilerParams(needs_layout_passes=False),
        out_type=jax.ShapeDtypeStruct((NW * NPAD,), jnp.float32),
        scratch_types=[
            pltpu.VMEM((CHUNK,), jnp.int32),
            pltpu.VMEM((NPAD,), jnp.float32),
        ],
    )
    def deg_kernel(dst1d, zeros1d, out, dst_v, acc):
        c = lax.axis_index("c")
        s = lax.axis_index("s")
        wid = c * 16 + s
        pltpu.sync_copy(zeros1d, acc)
        ones16 = jnp.ones((16,), jnp.float32)

        def body(j, _):
            base = (wid * NCHUNK + j) * CHUNK
            pltpu.sync_copy(dst1d.at[pl.ds(base, CHUNK)], dst_v)
            for k in range(CHUNK // 16):
                idx = dst_v[pl.ds(k * 16, 16)]
                plsc.addupdate_scatter(acc, [idx], ones16)
            return ()

        lax.fori_loop(0, NCHUNK, body, ())
        pltpu.sync_copy(acc, out.at[pl.ds(wid * NPAD, NPAD)])

    return deg_kernel


def _stage_a(x_ref, w_ref, dg_ref, dinv_ref, hs_ref):
    degs = lax.dot_general(dg_ref[...], jnp.ones((NW, 1), jnp.float32),
                           (((0,), (0,)), ((), ())),
                           preferred_element_type=jnp.float32, precision=lax.Precision.HIGHEST)
    dinv = lax.rsqrt(degs[:N] + 1.0)
    dinv_ref[...] = dinv
    h = jnp.dot(x_ref[...].astype(jnp.bfloat16),
                w_ref[...].astype(jnp.bfloat16),
                preferred_element_type=jnp.float32)
    hs_ref[...] = h * dinv


def _bn_relu(a, g, be):
    m = jnp.mean(a, axis=0, keepdims=True)
    v = jnp.mean((a - m) ** 2, axis=0, keepdims=True)
    return jnp.maximum(g * (a - m) * lax.rsqrt(v + EPS) + be, 0.0)


def _stage_b(p_ref, hs_ref, dinv_ref, b_ref, g_ref, be_ref, zs_ref):
    dinv = dinv_ref[...]
    a = dinv * (p_ref[0, :N, :] + p_ref[1, :N, :] + hs_ref[...]) + b_ref[...]
    z = _bn_relu(a, g_ref[...], be_ref[...])
    zs_ref[...] = z.astype(jnp.bfloat16).astype(jnp.float32) * dinv


def _stage_c(p_ref, zs_ref, dinv_ref, w_ref, b_ref, g_ref, be_ref, out_ref):
    dinv = dinv_ref[...]
    nmat = dinv * (p_ref[0, :N, :] + p_ref[1, :N, :] + zs_ref[...])
    a = jnp.dot(nmat, w_ref[...], preferred_element_type=jnp.float32,
                precision=lax.Precision.HIGHEST) + b_ref[...]
    z = _bn_relu(a, g_ref[...], be_ref[...])
    out_ref[...] = z.astype(jnp.bfloat16).astype(jnp.float32) * dinv


def _stage_d(p_ref, zs_ref, dinv_ref, w3_ref, b3_ref, g3_ref, be3_ref,
             batch_ref, zerof_ref, wl1_ref, bl1_ref, g4_ref, be4_ref,
             wl2_ref, bl2_ref, out_ref):
    dinv = dinv_ref[...]
    nmat = dinv * (p_ref[0, :N, :] + p_ref[1, :N, :] + zs_ref[...])
    a3 = jnp.dot(nmat, w3_ref[...], preferred_element_type=jnp.float32,
                 precision=lax.Precision.HIGHEST) + b3_ref[...]
    z3 = _bn_relu(a3, g3_ref[...], be3_ref[...])
    cols = lax.broadcasted_iota(jnp.int32, (N, 128), 1)
    onehot = (batch_ref[...] == cols).astype(jnp.float32)
    cnts = jnp.sum(onehot, axis=0, keepdims=True)
    scale = 1.0 / jnp.maximum(cnts + zerof_ref[...], 1.0)
    pooled = lax.dot_general(onehot * scale, z3,
                             (((0,), (0,)), ((), ())),
                             preferred_element_type=jnp.float32, precision=lax.Precision.HIGHEST)
    z4 = _bn_relu(
        jnp.dot(pooled.astype(jnp.bfloat16), wl1_ref[...].astype(jnp.bfloat16),
                preferred_element_type=jnp.float32)
        + bl1_ref[...], g4_ref[...], be4_ref[...])
    out_ref[...] = (jnp.dot(z4.astype(jnp.bfloat16),
                            wl2_ref[...].astype(jnp.bfloat16),
                            preferred_element_type=jnp.float32)
                    + bl2_ref[...])


def _tc(fn, out_shape, *args):
    return pl.pallas_call(fn, out_shape=out_shape)(*args)


def _padw(w, rows=None, cols=None):
    r = (rows or w.shape[0]) - w.shape[0]
    c = (cols or w.shape[1]) - w.shape[1]
    return jnp.pad(w, ((0, r), (0, c)))


def kernel(x, edge_index, batch, batch_size,
           W1, b1, g1, be1, W2, b2, g2, be2, W3, b3, g3, be3,
           Wl1, bl1, g4, be4, Wl2, bl2):
    f32 = jnp.float32
    pad = EPAD - E
    src = jnp.concatenate([edge_index[0], jnp.zeros((pad,), jnp.int32)])
    dst = jnp.concatenate([edge_index[1], jnp.full((pad,), N, jnp.int32)])

    zeros1d = jnp.zeros((NPAD,), f32)
    zeros2d = jnp.zeros((NPAD, D), f32)

    def vpad(v):
        return jnp.pad(v, (0, D - v.shape[0])).reshape(1, D)

    _DEG_DEBUG = False
    if _DEG_DEBUG:
        seg = jax.ops.segment_sum(jnp.ones((EPAD,), f32), dst,
                                  num_segments=NPAD)
        deg = jnp.zeros((NW, NPAD), f32).at[0].set(seg)
    else:
        deg = _make_deg()(dst, zeros1d).reshape(NW, NPAD)

    sd = jax.ShapeDtypeStruct
    dinv, hs1 = _tc(_stage_a, (sd((N, 1), f32), sd((N, D), f32)),
                    x, _padw(W1, cols=D), deg)

    _AGG_DEBUG = False
    if _AGG_DEBUG:
        def agg(table, src_, dst_, zeros_):
            half = EPAD // 2
            outs = []
            for hh in range(2):
                sl = slice(hh * half, (hh + 1) * half)
                outs.append(jax.ops.segment_sum(table[src_[sl]], dst_[sl],
                                                num_segments=NPAD))
            return jnp.stack(outs)
    else:
        agg = _make_agg()

    p1 = agg(hs1, src, dst, zeros2d)
    zs1 = _tc(_stage_b, sd((N, D), f32),
              p1, hs1, dinv, vpad(b1), vpad(g1), vpad(be1))

    p2 = agg(zs1, src, dst, zeros2d)
    w2b = _padw(W2.astype(jnp.bfloat16).astype(f32), rows=D)
    zs2 = _tc(_stage_c, sd((N, D), f32),
              p2, zs1, dinv, w2b, vpad(b2), vpad(g2), vpad(be2))

    p3 = agg(zs2, src, dst, zeros2d)
    zerof = (jnp.asarray(batch_size) - 128).astype(f32).reshape(1, 1)
    out = _tc(_stage_d, sd((128, 10), f32),
              p3, zs2, dinv, W3.astype(jnp.bfloat16).astype(f32),
              b3.reshape(1, -1), g3.reshape(1, -1),
              be3.reshape(1, -1), batch.reshape(N, 1), zerof,
              Wl1, bl1.reshape(1, -1), g4.reshape(1, -1), be4.reshape(1, -1),
              Wl2, bl2.reshape(1, -1))
    return out


# speedup vs baseline: 8.5002x; 1.2213x over previous
import functools

import jax
import jax.numpy as jnp
from jax import lax
from jax.experimental import pallas as pl
from jax.experimental.pallas import tpu as pltpu
from jax.experimental.pallas import tpu_sc as plsc

N = 10000
E = 320000
NPAD = 10112
RPT = NPAD // 16
NW = 32
CHUNK = 128
NCHUNK = 80
EPW = NCHUNK * CHUNK
EPAD = NW * EPW
D = 128
EPS = 1e-5

_MESH = dict(core_axis_name="c", subcore_axis_name="s")


@functools.lru_cache(maxsize=None)
def _make_agg():
    nbuf = 2
    half = NCHUNK // 2

    @functools.partial(
        pl.kernel,
        mesh=plsc.VectorSubcoreMesh(**_MESH),
        out_type=jax.ShapeDtypeStruct((2, NPAD, D), jnp.float32),
        scratch_types=[
            pltpu.VMEM((half, CHUNK), jnp.int32),
            pltpu.VMEM((half, CHUNK), jnp.int32),
            [pltpu.VMEM((CHUNK, D), jnp.float32)] * nbuf,
            pltpu.VMEM_SHARED((NPAD, D), jnp.float32),
            [pltpu.SemaphoreType.DMA] * nbuf,
        ],
    )
    def agg(table, src4d, dst4d, zeros, out, src_vv, dst_vv, rows, acc, sems):
        c = lax.axis_index("c")
        s = lax.axis_index("s")
        wid = c * 16 + s
        pltpu.sync_copy(zeros.at[pl.ds(s * RPT, RPT)],
                        acc.at[pl.ds(s * RPT, RPT)])
        plsc.subcore_barrier()

        def phase(ph, _):
            pltpu.sync_copy(src4d.at[wid, ph], src_vv)
            pltpu.sync_copy(dst4d.at[wid, ph], dst_vv)
            for b in range(nbuf):
                pltpu.async_copy(table.at[src_vv.at[b]], rows[b], sems[b])

            def outer(i, _):
                j0 = i * nbuf
                for b in range(nbuf):
                    j = j0 + b
                    pltpu.make_async_copy(table.at[src_vv.at[0]], rows[b],
                                          sems[b]).wait()
                    pltpu.sync_copy(rows[b], acc.at[dst_vv.at[j]], add=True)

                    @pl.when(j + nbuf < half)
                    def _():
                        pltpu.async_copy(table.at[src_vv.at[j + nbuf]],
                                         rows[b], sems[b])
                return ()

            lax.fori_loop(0, half // nbuf, outer, ())
            return ()

        lax.fori_loop(0, 2, phase, ())
        plsc.subcore_barrier()
        pltpu.sync_copy(acc.at[pl.ds(s * RPT, RPT)],
                        out.at[c, pl.ds(s * RPT, RPT)])

    return agg


@functools.lru_cache(maxsize=None)
def _make_deg():
    @functools.partial(
        pl.kernel,
        mesh=plsc.VectorSubcoreMesh(**_MESH),
        compiler_params=pltpu.CompilerParams(needs_layout_passes=False),
        out_type=jax.ShapeDtypeStruct((NW * NPAD,), jnp.float32),
        scratch_types=[
            pltpu.VMEM((CHUNK,), jnp.int32),
            pltpu.VMEM((NPAD,), jnp.float32),
        ],
    )
    def deg_kernel(dst1d, zeros1d, out, dst_v, acc):
        c = lax.axis_index("c")
        s = lax.axis_index("s")
        wid = c * 16 + s
        pltpu.sync_copy(zeros1d, acc)
        ones16 = jnp.ones((16,), jnp.float32)

        def body(j, _):
            base = (wid * NCHUNK + j) * CHUNK
            pltpu.sync_copy(dst1d.at[pl.ds(base, CHUNK)], dst_v)
            for k in range(CHUNK // 16):
                idx = dst_v[pl.ds(k * 16, 16)]
                plsc.addupdate_scatter(acc, [idx], ones16)
            return ()

        lax.fori_loop(0, NCHUNK, body, ())
        pltpu.sync_copy(acc, out.at[pl.ds(wid * NPAD, NPAD)])

    return deg_kernel


def _stage_a(x_ref, w_ref, dg_ref, dinv_ref, hs_ref):
    degs = lax.dot_general(dg_ref[...], jnp.ones((NW, 1), jnp.float32),
                           (((0,), (0,)), ((), ())),
                           preferred_element_type=jnp.float32, precision=lax.Precision.HIGHEST)
    dinv = lax.rsqrt(degs[:N] + 1.0)
    dinv_ref[...] = dinv
    h = jnp.dot(x_ref[...].astype(jnp.bfloat16),
                w_ref[...].astype(jnp.bfloat16),
                preferred_element_type=jnp.float32)
    hs_ref[...] = h * dinv


def _bn_relu(a, g, be):
    m = jnp.mean(a, axis=0, keepdims=True)
    v = jnp.mean((a - m) ** 2, axis=0, keepdims=True)
    return jnp.maximum(g * (a - m) * lax.rsqrt(v + EPS) + be, 0.0)


def _stage_b(p_ref, hs_ref, dinv_ref, b_ref, g_ref, be_ref, zs_ref):
    dinv = dinv_ref[...]
    a = dinv * (p_ref[0, :N, :] + p_ref[1, :N, :] + hs_ref[...]) + b_ref[...]
    z = _bn_relu(a, g_ref[...], be_ref[...])
    zs_ref[...] = z.astype(jnp.bfloat16).astype(jnp.float32) * dinv


def _stage_c(p_ref, zs_ref, dinv_ref, w_ref, b_ref, g_ref, be_ref, out_ref):
    dinv = dinv_ref[...]
    nmat = dinv * (p_ref[0, :N, :] + p_ref[1, :N, :] + zs_ref[...])
    a = jnp.dot(nmat, w_ref[...], preferred_element_type=jnp.float32,
                precision=lax.Precision.HIGHEST) + b_ref[...]
    z = _bn_relu(a, g_ref[...], be_ref[...])
    out_ref[...] = z.astype(jnp.bfloat16).astype(jnp.float32) * dinv


def _stage_d(p_ref, zs_ref, dinv_ref, w3_ref, b3_ref, g3_ref, be3_ref,
             batch_ref, zerof_ref, wl1_ref, bl1_ref, g4_ref, be4_ref,
             wl2_ref, bl2_ref, out_ref):
    dinv = dinv_ref[...]
    nmat = dinv * (p_ref[0, :N, :] + p_ref[1, :N, :] + zs_ref[...])
    a3 = jnp.dot(nmat, w3_ref[...], preferred_element_type=jnp.float32,
                 precision=lax.Precision.HIGHEST) + b3_ref[...]
    z3 = _bn_relu(a3, g3_ref[...], be3_ref[...])
    cols = lax.broadcasted_iota(jnp.int32, (N, 128), 1)
    onehot = (batch_ref[...] == cols).astype(jnp.float32)
    cnts = jnp.sum(onehot, axis=0, keepdims=True)
    scale = 1.0 / jnp.maximum(cnts + zerof_ref[...], 1.0)
    pooled = lax.dot_general(onehot * scale, z3,
                             (((0,), (0,)), ((), ())),
                             preferred_element_type=jnp.float32, precision=lax.Precision.HIGHEST)
    z4 = _bn_relu(
        jnp.dot(pooled.astype(jnp.bfloat16), wl1_ref[...].astype(jnp.bfloat16),
                preferred_element_type=jnp.float32)
        + bl1_ref[...], g4_ref[...], be4_ref[...])
    out_ref[...] = (jnp.dot(z4.astype(jnp.bfloat16),
                            wl2_ref[...].astype(jnp.bfloat16),
                            preferred_element_type=jnp.float32)
                    + bl2_ref[...])


def _tc(fn, out_shape, *args):
    return pl.pallas_call(fn, out_shape=out_shape)(*args)


def _padw(w, rows=None, cols=None):
    r = (rows or w.shape[0]) - w.shape[0]
    c = (cols or w.shape[1]) - w.shape[1]
    return jnp.pad(w, ((0, r), (0, c)))


def kernel(x, edge_index, batch, batch_size,
           W1, b1, g1, be1, W2, b2, g2, be2, W3, b3, g3, be3,
           Wl1, bl1, g4, be4, Wl2, bl2):
    f32 = jnp.float32
    pad = EPAD - E
    src = jnp.concatenate([edge_index[0], jnp.zeros((pad,), jnp.int32)])
    dst = jnp.concatenate([edge_index[1], jnp.full((pad,), N, jnp.int32)])

    zeros1d = jnp.zeros((NPAD,), f32)
    zeros2d = jnp.zeros((NPAD, D), f32)

    def vpad(v):
        return jnp.pad(v, (0, D - v.shape[0])).reshape(1, D)

    _DEG_DEBUG = False
    if _DEG_DEBUG:
        seg = jax.ops.segment_sum(jnp.ones((EPAD,), f32), dst,
                                  num_segments=NPAD)
        deg = jnp.zeros((NW, NPAD), f32).at[0].set(seg)
    else:
        deg = _make_deg()(dst, zeros1d).reshape(NW, NPAD)

    sd = jax.ShapeDtypeStruct
    dinv, hs1 = _tc(_stage_a, (sd((N, 1), f32), sd((N, D), f32)),
                    x, _padw(W1, cols=D), deg)

    _AGG_DEBUG = False
    if _AGG_DEBUG:
        def agg(table, src_, dst_, zeros_):
            half = EPAD // 2
            outs = []
            for hh in range(2):
                sl = slice(hh * half, (hh + 1) * half)
                outs.append(jax.ops.segment_sum(table[src_[sl]], dst_[sl],
                                                num_segments=NPAD))
            return jnp.stack(outs)
    else:
        agg = _make_agg()

    src3 = src.reshape(NW, 2, NCHUNK // 2, CHUNK)
    dst3 = dst.reshape(NW, 2, NCHUNK // 2, CHUNK)
    p1 = agg(hs1, src3, dst3, zeros2d)
    zs1 = _tc(_stage_b, sd((N, D), f32),
              p1, hs1, dinv, vpad(b1), vpad(g1), vpad(be1))

    p2 = agg(zs1, src3, dst3, zeros2d)
    w2b = _padw(W2.astype(jnp.bfloat16).astype(f32), rows=D)
    zs2 = _tc(_stage_c, sd((N, D), f32),
              p2, zs1, dinv, w2b, vpad(b2), vpad(g2), vpad(be2))

    p3 = agg(zs2, src3, dst3, zeros2d)
    zerof = (jnp.asarray(batch_size) - 128).astype(f32).reshape(1, 1)
    out = _tc(_stage_d, sd((128, 10), f32),
              p3, zs2, dinv, W3.astype(jnp.bfloat16).astype(f32),
              b3.reshape(1, -1), g3.reshape(1, -1),
              be3.reshape(1, -1), batch.reshape(N, 1), zerof,
              Wl1, bl1.reshape(1, -1), g4.reshape(1, -1), be4.reshape(1, -1),
              Wl2, bl2.reshape(1, -1))
    return out


# speedup vs baseline: 13.8678x; 1.6315x over previous
import functools

import jax
import jax.numpy as jnp
from jax import lax
from jax.experimental import pallas as pl
from jax.experimental.pallas import tpu as pltpu
from jax.experimental.pallas import tpu_sc as plsc

N = 10000
E = 320000
NPAD = 10112
RPT = NPAD // 16
NW = 32
CHUNK = 128
NCHUNK = 80
EPW = NCHUNK * CHUNK
EPAD = NW * EPW
D = 128
EPS = 1e-5

_MESH = dict(core_axis_name="c", subcore_axis_name="s")


@functools.lru_cache(maxsize=None)
def _make_agg(d=D, nbuf=2):
    half = NCHUNK // 2

    @functools.partial(
        pl.kernel,
        mesh=plsc.VectorSubcoreMesh(**_MESH),
        compiler_params=pltpu.CompilerParams(use_tc_tiling_on_sc=False),
        out_type=jax.ShapeDtypeStruct((2, NPAD, d), jnp.float32),
        scratch_types=[
            pltpu.VMEM((half, CHUNK), jnp.int32),
            pltpu.VMEM((half, CHUNK), jnp.int32),
            [pltpu.VMEM((CHUNK, d), jnp.float32)] * nbuf,
            pltpu.VMEM_SHARED((NPAD, d), jnp.float32),
            [pltpu.SemaphoreType.DMA] * nbuf,
        ],
    )
    def agg(table, src4d, dst4d, zeros, out, src_vv, dst_vv, rows, acc, sems):
        c = lax.axis_index("c")
        s = lax.axis_index("s")
        wid = c * 16 + s
        pltpu.sync_copy(zeros.at[pl.ds(s * RPT, RPT)],
                        acc.at[pl.ds(s * RPT, RPT)])
        plsc.subcore_barrier()

        def phase(ph, _):
            pltpu.sync_copy(src4d.at[wid, ph], src_vv)
            pltpu.sync_copy(dst4d.at[wid, ph], dst_vv)
            for b in range(nbuf):
                pltpu.async_copy(table.at[src_vv.at[b]], rows[b], sems[b])

            def outer(i, _):
                j0 = i * nbuf
                for b in range(nbuf):
                    j = j0 + b
                    pltpu.make_async_copy(table.at[src_vv.at[0]], rows[b],
                                          sems[b]).wait()
                    pltpu.sync_copy(rows[b], acc.at[dst_vv.at[j]], add=True)

                    @pl.when(j + nbuf < half)
                    def _():
                        pltpu.async_copy(table.at[src_vv.at[j + nbuf]],
                                         rows[b], sems[b])
                return ()

            lax.fori_loop(0, half // nbuf, outer, ())
            return ()

        lax.fori_loop(0, 2, phase, ())
        plsc.subcore_barrier()
        pltpu.sync_copy(acc.at[pl.ds(s * RPT, RPT)],
                        out.at[c, pl.ds(s * RPT, RPT)])

    return agg


@functools.lru_cache(maxsize=None)
def _make_deg():
    @functools.partial(
        pl.kernel,
        mesh=plsc.VectorSubcoreMesh(**_MESH),
        compiler_params=pltpu.CompilerParams(needs_layout_passes=False),
        out_type=jax.ShapeDtypeStruct((NW * NPAD,), jnp.float32),
        scratch_types=[
            pltpu.VMEM((CHUNK,), jnp.int32),
            pltpu.VMEM((NPAD,), jnp.float32),
        ],
    )
    def deg_kernel(dst1d, zeros1d, out, dst_v, acc):
        c = lax.axis_index("c")
        s = lax.axis_index("s")
        wid = c * 16 + s
        pltpu.sync_copy(zeros1d, acc)
        ones16 = jnp.ones((16,), jnp.float32)

        def body(j, _):
            base = (wid * NCHUNK + j) * CHUNK
            pltpu.sync_copy(dst1d.at[pl.ds(base, CHUNK)], dst_v)
            for k in range(CHUNK // 16):
                idx = dst_v[pl.ds(k * 16, 16)]
                plsc.addupdate_scatter(acc, [idx], ones16)
            return ()

        lax.fori_loop(0, NCHUNK, body, ())
        pltpu.sync_copy(acc, out.at[pl.ds(wid * NPAD, NPAD)])

    return deg_kernel


def _stage_a(x_ref, w_ref, dg_ref, dinv_ref, hs_ref):
    degs = lax.dot_general(dg_ref[...], jnp.ones((NW, 1), jnp.float32),
                           (((0,), (0,)), ((), ())),
                           preferred_element_type=jnp.float32, precision=lax.Precision.HIGHEST)
    dinv = lax.rsqrt(degs[:N] + 1.0)
    dinv_ref[...] = dinv
    h = jnp.dot(x_ref[...].astype(jnp.bfloat16),
                w_ref[...].astype(jnp.bfloat16),
                preferred_element_type=jnp.float32)
    hs_ref[...] = h * dinv


def _bn_relu(a, g, be):
    m = jnp.mean(a, axis=0, keepdims=True)
    v = jnp.mean((a - m) ** 2, axis=0, keepdims=True)
    return jnp.maximum(g * (a - m) * lax.rsqrt(v + EPS) + be, 0.0)


def _stage_b(p_ref, hs_ref, dinv_ref, b_ref, g_ref, be_ref, zs_ref):
    dinv = dinv_ref[...]
    a = dinv * (p_ref[0, :N, :] + p_ref[1, :N, :] + hs_ref[...]) + b_ref[...]
    z = _bn_relu(a, g_ref[...], be_ref[...])
    zs_ref[...] = z.astype(jnp.bfloat16).astype(jnp.float32) * dinv


def _stage_c(p_ref, zs_ref, dinv_ref, w_ref, b_ref, g_ref, be_ref, out_ref):
    dinv = dinv_ref[...]
    nmat = dinv * (p_ref[0, :N, :] + p_ref[1, :N, :] + zs_ref[...])
    a = jnp.dot(nmat, w_ref[...], preferred_element_type=jnp.float32,
                precision=lax.Precision.HIGHEST) + b_ref[...]
    z = _bn_relu(a, g_ref[...], be_ref[...])
    out_ref[...] = z.astype(jnp.bfloat16).astype(jnp.float32) * dinv


def _stage_d(p_ref, zs_ref, dinv_ref, w3_ref, b3_ref, g3_ref, be3_ref,
             batch_ref, zerof_ref, wl1_ref, bl1_ref, g4_ref, be4_ref,
             wl2_ref, bl2_ref, out_ref):
    dinv = dinv_ref[...]
    nmat = dinv * (p_ref[0, :N, :] + p_ref[1, :N, :] + zs_ref[...])
    a3 = jnp.dot(nmat, w3_ref[...], preferred_element_type=jnp.float32,
                 precision=lax.Precision.HIGHEST) + b3_ref[...]
    z3 = _bn_relu(a3, g3_ref[...], be3_ref[...])
    cols = lax.broadcasted_iota(jnp.int32, (N, 128), 1)
    onehot = (batch_ref[...] == cols).astype(jnp.float32)
    cnts = jnp.sum(onehot, axis=0, keepdims=True)
    scale = 1.0 / jnp.maximum(cnts + zerof_ref[...], 1.0)
    pooled = lax.dot_general(onehot * scale, z3,
                             (((0,), (0,)), ((), ())),
                             preferred_element_type=jnp.float32, precision=lax.Precision.HIGHEST)
    z4 = _bn_relu(
        jnp.dot(pooled.astype(jnp.bfloat16), wl1_ref[...].astype(jnp.bfloat16),
                preferred_element_type=jnp.float32)
        + bl1_ref[...], g4_ref[...], be4_ref[...])
    out_ref[...] = (jnp.dot(z4.astype(jnp.bfloat16),
                            wl2_ref[...].astype(jnp.bfloat16),
                            preferred_element_type=jnp.float32)
                    + bl2_ref[...])


def _tc(fn, out_shape, *args):
    return pl.pallas_call(fn, out_shape=out_shape)(*args)


def kernel(x, edge_index, batch, batch_size,
           W1, b1, g1, be1, W2, b2, g2, be2, W3, b3, g3, be3,
           Wl1, bl1, g4, be4, Wl2, bl2):
    f32 = jnp.float32
    pad = EPAD - E
    src = jnp.concatenate([edge_index[0], jnp.zeros((pad,), jnp.int32)])
    dst = jnp.concatenate([edge_index[1], jnp.full((pad,), N, jnp.int32)])

    zeros1d = jnp.zeros((NPAD,), f32)
    zeros64 = jnp.zeros((NPAD, 64), f32)
    zeros128 = jnp.zeros((NPAD, 128), f32)

    deg = _make_deg()(dst, zeros1d).reshape(NW, NPAD)

    sd = jax.ShapeDtypeStruct
    dinv, hs1 = _tc(_stage_a, (sd((N, 1), f32), sd((N, 64), f32)),
                    x, W1, deg)

    agg64 = _make_agg(64, 4)
    agg128 = _make_agg(128, 2)

    src3 = src.reshape(NW, 2, NCHUNK // 2, CHUNK)
    dst3 = dst.reshape(NW, 2, NCHUNK // 2, CHUNK)
    p1 = agg64(hs1, src3, dst3, zeros64)
    zs1 = _tc(_stage_b, sd((N, 64), f32),
              p1, hs1, dinv, b1.reshape(1, -1), g1.reshape(1, -1),
              be1.reshape(1, -1))

    p2 = agg64(zs1, src3, dst3, zeros64)
    w2b = W2.astype(jnp.bfloat16).astype(f32)
    zs2 = _tc(_stage_c, sd((N, 128), f32),
              p2, zs1, dinv, w2b, b2.reshape(1, -1), g2.reshape(1, -1),
              be2.reshape(1, -1))

    p3 = agg128(zs2, src3, dst3, zeros128)
    zerof = (jnp.asarray(batch_size) - 128).astype(f32).reshape(1, 1)
    out = _tc(_stage_d, sd((128, 10), f32),
              p3, zs2, dinv, W3.astype(jnp.bfloat16).astype(f32),
              b3.reshape(1, -1), g3.reshape(1, -1),
              be3.reshape(1, -1), batch.reshape(N, 1), zerof,
              Wl1, bl1.reshape(1, -1), g4.reshape(1, -1), be4.reshape(1, -1),
              Wl2, bl2.reshape(1, -1))
    return out
